# baseline (speedup 1.0000x reference)
import functools
import math

import jax
import jax.numpy as jnp
from jax.experimental import pallas as pl
from jax.experimental.pallas import tpu as pltpu

B, S, D = 16, 512, 2048
K = 16
H = 1024
SF = 128
SH = 256
MULT = 1.0


def _pool_kernel(seq_ref, x_ref, pooled_ref):
    b = pl.program_id(0)
    x = x_ref[0]
    n = seq_ref[b]
    row = jax.lax.broadcasted_iota(jnp.int32, (S, D), 0)
    mask = (row < n).astype(jnp.float32)
    denom = jnp.maximum(n.astype(jnp.float32), 1.0)
    pooled_ref[0] = jnp.sum(x * mask, axis=0, keepdims=True) / denom


def _heads_kernel(pooled_ref, w1_ref, b1_ref, t_ref, w3_ref, b3_ref,
                  h_ref, hs_ref):
    pooled = pooled_ref[...][:, 0, :]
    pre = jnp.dot(pooled, w1_ref[...],
                  preferred_element_type=jnp.float32) + b1_ref[...]
    h_ref[...] = 0.5 * pre * (1.0 + jax.lax.erf(pre * (2.0 ** -0.5)))
    t = t_ref[...]
    half = SF // 2
    idx = jax.lax.broadcasted_iota(jnp.int32, (B, half), 1).astype(jnp.float32)
    freqs = jnp.exp((-math.log(10000.0) / half) * idx)
    angles = t * freqs
    feat = jnp.concatenate([jnp.cos(angles), jnp.sin(angles)], axis=1)
    pre_s = jnp.dot(feat, w3_ref[...],
                    preferred_element_type=jnp.float32) + b3_ref[...]
    hs_ref[...] = pre_s * jax.nn.sigmoid(pre_s)


def _splice_kernel(emb_ref, h_ref, hs_ref, w2_ref, b2_ref, w4_ref,
                   b4_ref, slot_ref, out_ref,
                   vbuf0, vbuf1, copy_sem, vsem0, vsem1):
    j = pl.program_id(0)
    vbufs = (vbuf0, vbuf1)
    vsems = (vsem0, vsem1)

    def copy_desc(b):
        return pltpu.make_async_copy(
            emb_ref.at[b, : S - K], out_ref.at[b, : S - K], copy_sem)

    def val_desc(idx, par):
        return pltpu.make_async_copy(
            vbufs[par],
            out_ref.at[:, pl.ds(S - K + idx, 1), :],
            vsems[par])

    @pl.when(j == 0)
    def _():
        for b in range(B):
            copy_desc(b).start()

    val = jnp.dot(h_ref[...], w2_ref[...],
                  preferred_element_type=jnp.float32)
    val = val + jnp.dot(hs_ref[...], w4_ref[...],
                        preferred_element_type=jnp.float32)
    val = val + b2_ref[...] + b4_ref[...] + slot_ref[0]
    val = val * MULT

    for par in range(2):
        @pl.when((j >= 2) & (j % 2 == par))
        def _():
            val_desc(j - 2, par).wait()

        @pl.when(j % 2 == par)
        def _():
            vbufs[par][...] = val.reshape(B, 1, D)
            val_desc(j, par).start()

    @pl.when(j == K - 1)
    def _():
        val_desc(j - 1, (K - 2) % 2).wait()
        val_desc(j, (K - 1) % 2).wait()
        for b in range(B):
            copy_desc(b).wait()


def kernel(crossattn_emb, crossattn_seqlens, timesteps, W1, b1, W2, b2,
           slot_embed, W3, b3, W4, b4):
    f32 = jnp.float32

    pooled = pl.pallas_call(
        _pool_kernel,
        grid=(B,),
        in_specs=[
            pl.BlockSpec(memory_space=pltpu.SMEM),
            pl.BlockSpec((1, S, D), lambda b: (b, 0, 0)),
        ],
        out_specs=pl.BlockSpec((1, 1, D), lambda b: (b, 0, 0)),
        out_shape=jax.ShapeDtypeStruct((B, 1, D), f32),
    )(crossattn_seqlens.astype(jnp.int32), crossattn_emb)

    h, hs = pl.pallas_call(
        _heads_kernel,
        in_specs=[
            pl.BlockSpec((B, 1, D), lambda: (0, 0, 0)),
            pl.BlockSpec((D, H), lambda: (0, 0)),
            pl.BlockSpec((1, H), lambda: (0, 0)),
            pl.BlockSpec((B, 1), lambda: (0, 0)),
            pl.BlockSpec((SF, SH), lambda: (0, 0)),
            pl.BlockSpec((1, SH), lambda: (0, 0)),
        ],
        out_specs=[
            pl.BlockSpec((B, H), lambda: (0, 0)),
            pl.BlockSpec((B, SH), lambda: (0, 0)),
        ],
        out_shape=[
            jax.ShapeDtypeStruct((B, H), f32),
            jax.ShapeDtypeStruct((B, SH), f32),
        ],
    )(pooled, W1, b1.reshape(1, H), timesteps.reshape(B, 1).astype(f32),
      W3, b3.reshape(1, SH))

    out = pl.pallas_call(
        _splice_kernel,
        grid=(K,),
        in_specs=[
            pl.BlockSpec(memory_space=pltpu.HBM),
            pl.BlockSpec((B, H), lambda j: (0, 0)),
            pl.BlockSpec((B, SH), lambda j: (0, 0)),
            pl.BlockSpec((H, D), lambda j: (0, j)),
            pl.BlockSpec((1, D), lambda j: (0, j)),
            pl.BlockSpec((SH, D), lambda j: (0, j)),
            pl.BlockSpec((1, D), lambda j: (0, j)),
            pl.BlockSpec((1, 1, D), lambda j: (j, 0, 0)),
        ],
        out_specs=pl.BlockSpec(memory_space=pltpu.HBM),
        out_shape=jax.ShapeDtypeStruct((B, S, D), f32),
        scratch_shapes=[
            pltpu.VMEM((B, 1, D), f32),
            pltpu.VMEM((B, 1, D), f32),
            pltpu.SemaphoreType.DMA,
            pltpu.SemaphoreType.DMA,
            pltpu.SemaphoreType.DMA,
        ],
    )(crossattn_emb, h, hs, W2, b2.reshape(1, K * D), W4,
      b4.reshape(1, K * D), slot_embed.reshape(K, 1, D))

    return out


# speedup vs baseline: 18.7130x; 18.7130x over previous
import functools
import math

import jax
import jax.numpy as jnp
from jax.experimental import pallas as pl
from jax.experimental.pallas import tpu as pltpu

B, S, D = 16, 512, 2048
K = 16
H = 1024
SF = 128
SH = 256
MULT = 1.0


def _pool_copy_kernel(seq_ref, x_ref, out_ref, pooled_ref):
    b = pl.program_id(0)
    x = x_ref[0]
    n = seq_ref[b]
    row = jax.lax.broadcasted_iota(jnp.int32, (S, D), 0)
    mask = (row < n).astype(jnp.float32)
    denom = jnp.maximum(n.astype(jnp.float32), 1.0)
    pooled_ref[0] = jnp.sum(x * mask, axis=0, keepdims=True) / denom
    out_ref[0] = x[: S - K]


def _heads_kernel(pooled_ref, w1_ref, b1_ref, t_ref, w3_ref, b3_ref,
                  h_ref, hs_ref):
    pooled = pooled_ref[...][:, 0, :]
    pre = jnp.dot(pooled, w1_ref[...],
                  preferred_element_type=jnp.float32) + b1_ref[...]
    h_ref[...] = 0.5 * pre * (1.0 + jax.lax.erf(pre * (2.0 ** -0.5)))
    t = t_ref[...]
    half = SF // 2
    idx = jax.lax.broadcasted_iota(jnp.int32, (B, half), 1).astype(jnp.float32)
    freqs = jnp.exp((-math.log(10000.0) / half) * idx)
    angles = t * freqs
    feat = jnp.concatenate([jnp.cos(angles), jnp.sin(angles)], axis=1)
    pre_s = jnp.dot(feat, w3_ref[...],
                    preferred_element_type=jnp.float32) + b3_ref[...]
    hs_ref[...] = pre_s * jax.nn.sigmoid(pre_s)


_HALF = D // 2


def _splice_kernel(out_in_ref, h_ref, hs_ref, w2_ref, b2_ref, w4_ref,
                   b4_ref, slot_ref, out_ref):
    del out_in_ref
    j = pl.program_id(0)
    k = j // 2
    col = (j % 2) * _HALF
    val = jnp.dot(h_ref[...], w2_ref[...],
                  preferred_element_type=jnp.float32)
    val = val + jnp.dot(hs_ref[...], w4_ref[...],
                        preferred_element_type=jnp.float32)
    val = val + b2_ref[...] + b4_ref[...] + slot_ref[0]
    out_ref[:, pl.ds(k, 1), pl.ds(col, _HALF)] = (val * MULT).reshape(
        B, 1, _HALF)


def kernel(crossattn_emb, crossattn_seqlens, timesteps, W1, b1, W2, b2,
           slot_embed, W3, b3, W4, b4):
    f32 = jnp.float32

    copy_out, pooled = pl.pallas_call(
        _pool_copy_kernel,
        grid=(B,),
        in_specs=[
            pl.BlockSpec(memory_space=pltpu.SMEM),
            pl.BlockSpec((1, S, D), lambda b: (b, 0, 0)),
        ],
        out_specs=[
            pl.BlockSpec((1, S - K, D), lambda b: (b, 0, 0)),
            pl.BlockSpec((1, 1, D), lambda b: (b, 0, 0)),
        ],
        out_shape=[
            jax.ShapeDtypeStruct((B, S, D), f32),
            jax.ShapeDtypeStruct((B, 1, D), f32),
        ],
    )(crossattn_seqlens.astype(jnp.int32), crossattn_emb)

    h, hs = pl.pallas_call(
        _heads_kernel,
        in_specs=[
            pl.BlockSpec((B, 1, D), lambda: (0, 0, 0)),
            pl.BlockSpec((D, H), lambda: (0, 0)),
            pl.BlockSpec((1, H), lambda: (0, 0)),
            pl.BlockSpec((B, 1), lambda: (0, 0)),
            pl.BlockSpec((SF, SH), lambda: (0, 0)),
            pl.BlockSpec((1, SH), lambda: (0, 0)),
        ],
        out_specs=[
            pl.BlockSpec((B, H), lambda: (0, 0)),
            pl.BlockSpec((B, SH), lambda: (0, 0)),
        ],
        out_shape=[
            jax.ShapeDtypeStruct((B, H), f32),
            jax.ShapeDtypeStruct((B, SH), f32),
        ],
    )(pooled, W1, b1.reshape(1, H), timesteps.reshape(B, 1).astype(f32),
      W3, b3.reshape(1, SH))

    out = pl.pallas_call(
        _splice_kernel,
        grid=(2 * K,),
        in_specs=[
            pl.BlockSpec(memory_space=pltpu.HBM),
            pl.BlockSpec((B, H), lambda j: (0, 0)),
            pl.BlockSpec((B, SH), lambda j: (0, 0)),
            pl.BlockSpec((H, _HALF), lambda j: (0, j)),
            pl.BlockSpec((1, _HALF), lambda j: (0, j)),
            pl.BlockSpec((SH, _HALF), lambda j: (0, j)),
            pl.BlockSpec((1, _HALF), lambda j: (0, j)),
            pl.BlockSpec((1, 1, _HALF), lambda j: (j // 2, 0, j % 2)),
        ],
        out_specs=pl.BlockSpec((B, K, D), lambda j: (0, (S - K) // K, 0)),
        out_shape=jax.ShapeDtypeStruct((B, S, D), f32),
        input_output_aliases={0: 0},
    )(copy_out, h, hs, W2, b2.reshape(1, K * D), W4, b4.reshape(1, K * D),
      slot_embed.reshape(K, 1, D))

    return out


# speedup vs baseline: 19.3417x; 1.0336x over previous
import math

import jax
import jax.numpy as jnp
from jax.experimental import pallas as pl
from jax.experimental.pallas import tpu as pltpu

B, S, D = 16, 512, 2048
K = 16
H = 1024
SF = 128
SH = 256
MULT = 1.0


def _pool_heads_kernel(seq_ref, x_ref, w1_ref, b1_ref, t_ref, w3_ref,
                       b3_ref, out_ref, h_ref, hs_ref, pooled_s):
    b = pl.program_id(0)
    x = x_ref[0]
    n = seq_ref[b]
    row = jax.lax.broadcasted_iota(jnp.int32, (S, D), 0)
    mask = (row < n).astype(jnp.float32)
    denom = jnp.maximum(n.astype(jnp.float32), 1.0)
    pooled_s[pl.ds(b, 1), :] = jnp.sum(x * mask, axis=0, keepdims=True) / denom
    out_ref[0] = x[: S - K]

    @pl.when(b == B - 1)
    def _():
        pooled = pooled_s[...]
        pre = jnp.dot(pooled, w1_ref[...],
                      preferred_element_type=jnp.float32) + b1_ref[...]
        h_ref[...] = 0.5 * pre * (1.0 + jax.lax.erf(pre * (2.0 ** -0.5)))
        t = t_ref[...]
        half = SF // 2
        idx = jax.lax.broadcasted_iota(
            jnp.int32, (B, half), 1).astype(jnp.float32)
        freqs = jnp.exp((-math.log(10000.0) / half) * idx)
        angles = t * freqs
        feat = jnp.concatenate([jnp.cos(angles), jnp.sin(angles)], axis=1)
        pre_s = jnp.dot(feat, w3_ref[...],
                        preferred_element_type=jnp.float32) + b3_ref[...]
        hs_ref[...] = pre_s * jax.nn.sigmoid(pre_s)


def _splice_kernel(out_in_ref, h_ref, hs_ref, w2_ref, b2_ref, w4_ref,
                   b4_ref, slot_ref, out_ref):
    del out_in_ref
    j = pl.program_id(0)
    val = jnp.dot(h_ref[...], w2_ref[...],
                  preferred_element_type=jnp.float32)
    val = val + jnp.dot(hs_ref[...], w4_ref[...],
                        preferred_element_type=jnp.float32)
    val = val + b2_ref[...] + b4_ref[...] + slot_ref[0]
    out_ref[:, j, :] = val * MULT


def kernel(crossattn_emb, crossattn_seqlens, timesteps, W1, b1, W2, b2,
           slot_embed, W3, b3, W4, b4):
    f32 = jnp.float32

    copy_out, h, hs = pl.pallas_call(
        _pool_heads_kernel,
        grid=(B,),
        in_specs=[
            pl.BlockSpec(memory_space=pltpu.SMEM),
            pl.BlockSpec((1, S, D), lambda b: (b, 0, 0)),
            pl.BlockSpec((D, H), lambda b: (0, 0)),
            pl.BlockSpec((1, H), lambda b: (0, 0)),
            pl.BlockSpec((B, 1), lambda b: (0, 0)),
            pl.BlockSpec((SF, SH), lambda b: (0, 0)),
            pl.BlockSpec((1, SH), lambda b: (0, 0)),
        ],
        out_specs=[
            pl.BlockSpec((1, S - K, D), lambda b: (b, 0, 0)),
            pl.BlockSpec((B, H), lambda b: (0, 0)),
            pl.BlockSpec((B, SH), lambda b: (0, 0)),
        ],
        out_shape=[
            jax.ShapeDtypeStruct((B, S, D), f32),
            jax.ShapeDtypeStruct((B, H), f32),
            jax.ShapeDtypeStruct((B, SH), f32),
        ],
        scratch_shapes=[pltpu.VMEM((B, D), f32)],
    )(crossattn_seqlens.astype(jnp.int32), crossattn_emb, W1,
      b1.reshape(1, H), timesteps.reshape(B, 1).astype(f32), W3,
      b3.reshape(1, SH))

    out = pl.pallas_call(
        _splice_kernel,
        grid=(K,),
        in_specs=[
            pl.BlockSpec(memory_space=pltpu.HBM),
            pl.BlockSpec((B, H), lambda j: (0, 0)),
            pl.BlockSpec((B, SH), lambda j: (0, 0)),
            pl.BlockSpec((H, D), lambda j: (0, j)),
            pl.BlockSpec((1, D), lambda j: (0, j)),
            pl.BlockSpec((SH, D), lambda j: (0, j)),
            pl.BlockSpec((1, D), lambda j: (0, j)),
            pl.BlockSpec((1, 1, D), lambda j: (j, 0, 0)),
        ],
        out_specs=pl.BlockSpec((B, K, D), lambda j: (0, (S - K) // K, 0)),
        out_shape=jax.ShapeDtypeStruct((B, S, D), f32),
        input_output_aliases={0: 0},
    )(copy_out, h, hs, W2, b2.reshape(1, K * D), W4, b4.reshape(1, K * D),
      slot_embed.reshape(K, 1, D))

    return out
